# Initial kernel scaffold; baseline (speedup 1.0000x reference)
import jax
import jax.numpy as jnp
from jax.experimental import pallas as pl


def kernel(hidden_state, obs1, obs2, W, b):
    raise NotImplementedError("write your pallas kernel here")


# speedup vs baseline: 67.8954x; 67.8954x over previous
import jax
import jax.numpy as jnp
from jax.experimental import pallas as pl

N = 4096
NG = 6
NB = NG * NG
NB_PAD = 40
HD = 128
BI = 128
BJ = 512
NCHUNK = N // BJ


def _pool_kernel(obs_i_ref, obs_t_ref, wt_ref, b_ref, out_ref):
    pid = pl.program_id(0)
    xi = obs_i_ref[:, 0:1]
    yi = obs_i_ref[:, 1:2]
    i_glob = pid * BI + jax.lax.broadcasted_iota(jnp.int32, (BI, 1), 0)

    occ_cols = [jnp.zeros((BI, 1), jnp.float32) for _ in range(NB)]
    for c in range(NCHUNK):
        xj = obs_t_ref[0:1, c * BJ:(c + 1) * BJ]
        yj = obs_t_ref[1:2, c * BJ:(c + 1) * BJ]
        ox = (xj - xi) + (NG / 2.0)
        oy = (yj - yi) + (NG / 2.0)
        j_glob = c * BJ + jax.lax.broadcasted_iota(jnp.int32, (1, BJ), 1)
        valid = ((ox >= 0.0) & (ox < float(NG)) &
                 (oy >= 0.0) & (oy < float(NG)) &
                 (i_glob != j_glob))
        xb = jnp.floor(jnp.clip(ox, 0.0, NG - 1.0)).astype(jnp.int32)
        yb = jnp.floor(jnp.clip(oy, 0.0, NG - 1.0)).astype(jnp.int32)
        binv = jnp.where(valid, xb * NG + yb, -1)
        for k in range(NB):
            hit = jnp.where(binv == k, 1.0, 0.0)
            occ_cols[k] = occ_cols[k] + jnp.sum(hit, axis=1, keepdims=True)

    occ_cols += [jnp.zeros((BI, 1), jnp.float32)] * (NB_PAD - NB)
    occ = jnp.concatenate(occ_cols, axis=1)
    out_ref[...] = (
        jnp.dot(occ, wt_ref[...], preferred_element_type=jnp.float32)
        + b_ref[...]
    )


@jax.jit
def kernel(hidden_state, obs1, obs2, W, b):
    del hidden_state, obs1
    obs_t = obs2.T
    wt = jnp.zeros((NB_PAD, HD), jnp.float32).at[:NB].set(W.T)
    out = pl.pallas_call(
        _pool_kernel,
        grid=(N // BI,),
        in_specs=[
            pl.BlockSpec((BI, 2), lambda i: (i, 0)),
            pl.BlockSpec((2, N), lambda i: (0, 0)),
            pl.BlockSpec((NB_PAD, HD), lambda i: (0, 0)),
            pl.BlockSpec((1, HD), lambda i: (0, 0)),
        ],
        out_specs=pl.BlockSpec((BI, HD), lambda i: (i, 0)),
        out_shape=jax.ShapeDtypeStruct((N, HD), jnp.float32),
    )(obs2, obs_t, wt, b.reshape(1, HD))
    return out


# speedup vs baseline: 118.1649x; 1.7404x over previous
import dataclasses
import jax
import jax.numpy as jnp
from jax import lax
from jax.experimental import pallas as pl
from jax.experimental.pallas import tpu as pltpu
from jax.experimental.pallas import tpu_sc as plsc

N = 4096
NG = 6
NB = NG * NG
ROWW = 40
HD = 128
NC = 2
NS = 16
NW = NC * NS
IPW = N // NW
LN = 16
BM = 512

_mesh = plsc.VectorSubcoreMesh(core_axis_name="core", subcore_axis_name="subcore")

_cp = pltpu.CompilerParams()
if "needs_layout_passes" in pltpu.CompilerParams.__dataclass_fields__:
    _cp = dataclasses.replace(_cp, needs_layout_passes=False)


def _hist_body(xs_hbm, ys_hbm, occ_hbm, xs_v, ys_v, occ_v, sem):
    cid = lax.axis_index("core")
    sid = lax.axis_index("subcore")
    wid = cid * NS + sid
    base_i = wid * IPW

    copy_x = pltpu.make_async_copy(xs_hbm, xs_v, sem)
    copy_x.start()
    copy_y = pltpu.make_async_copy(ys_hbm, ys_v, sem)
    copy_y.start()

    zero16 = jnp.zeros((LN,), jnp.float32)

    @pl.loop(0, IPW * ROWW, step=LN)
    def _(k):
        occ_v[pl.ds(k, LN)] = zero16

    copy_x.wait()
    copy_y.wait()

    lane = lax.broadcasted_iota(jnp.int32, (LN,), 0)
    ones = jnp.ones((LN,), jnp.float32)

    @pl.loop(0, IPW, step=LN)
    def _(ic):
        xi = xs_v[pl.ds(base_i + ic, LN)]
        yi = ys_v[pl.ds(base_i + ic, LN)]
        rowbase = (ic + lane) * ROWW

        @pl.loop(0, N, step=LN)
        def _(jc):
            xjv = xs_v[pl.ds(jc, LN)]
            yjv = ys_v[pl.ds(jc, LN)]
            for jj in range(LN):
                ox = (xjv[jj] - xi) + (NG / 2.0)
                oy = (yjv[jj] - yi) + (NG / 2.0)
                m = ((jnp.minimum(ox, oy) >= 0.0) &
                     (jnp.maximum(ox, oy) < float(NG)))
                xb = ox.astype(jnp.int32)
                yb = oy.astype(jnp.int32)
                idx = rowbase + (xb * NG + yb)
                plsc.addupdate_scatter(occ_v, [idx], ones, mask=m)

        selfm = (xi * 0.0 == 0.0) & (yi * 0.0 == 0.0)
        plsc.addupdate_scatter(occ_v, [rowbase + (3 * NG + 3)], -ones, mask=selfm)

    pltpu.sync_copy(occ_v, occ_hbm.at[pl.ds(base_i * ROWW, IPW * ROWW)])


def _occupancy_sc(xs, ys):
    k = pl.kernel(
        _hist_body,
        out_type=jax.ShapeDtypeStruct((N * ROWW,), jnp.float32),
        mesh=_mesh,
        scratch_types=[
            pltpu.VMEM((N,), jnp.float32),
            pltpu.VMEM((N,), jnp.float32),
            pltpu.VMEM((IPW * ROWW,), jnp.float32),
            pltpu.SemaphoreType.DMA,
        ],
        compiler_params=_cp,
    )
    return k(xs, ys)


def _embed_kernel(occ_ref, wt_ref, b_ref, out_ref):
    out_ref[...] = (
        jnp.dot(occ_ref[...], wt_ref[...], preferred_element_type=jnp.float32)
        + b_ref[...]
    )


@jax.jit
def kernel(hidden_state, obs1, obs2, W, b):
    del hidden_state, obs1
    xs = obs2[:, 0]
    ys = obs2[:, 1]
    occ = _occupancy_sc(xs, ys).reshape(N, ROWW)
    wt = jnp.zeros((ROWW, HD), jnp.float32).at[:NB].set(W.T)
    out = pl.pallas_call(
        _embed_kernel,
        grid=(N // BM,),
        in_specs=[
            pl.BlockSpec((BM, ROWW), lambda i: (i, 0)),
            pl.BlockSpec((ROWW, HD), lambda i: (0, 0)),
            pl.BlockSpec((1, HD), lambda i: (0, 0)),
        ],
        out_specs=pl.BlockSpec((BM, HD), lambda i: (i, 0)),
        out_shape=jax.ShapeDtypeStruct((N, HD), jnp.float32),
    )(occ, wt, b.reshape(1, HD))
    return out


# speedup vs baseline: 147.7160x; 1.2501x over previous
import dataclasses
import jax
import jax.numpy as jnp
from jax import lax
from jax.experimental import pallas as pl
from jax.experimental.pallas import tpu as pltpu
from jax.experimental.pallas import tpu_sc as plsc

N = 4096
NG = 6
NB = NG * NG
ROWW = 40
HD = 128
NC = 2
NS = 16
NW = NC * NS
LN = 16
BM = 512

SPLIT = 1536
NSC = N - SPLIT
IPW = NSC // NW

BI = 128
BJ = 512
NCHUNK = N // BJ

_cp = pltpu.CompilerParams()
if "needs_layout_passes" in pltpu.CompilerParams.__dataclass_fields__:
    _cp = dataclasses.replace(_cp, needs_layout_passes=False)


def _pool_kernel(obs_i_ref, obs_t_ref, wt_ref, b_ref, out_ref):
    pid = pl.program_id(0)
    xi = obs_i_ref[:, 0:1]
    yi = obs_i_ref[:, 1:2]
    i_glob = pid * BI + jax.lax.broadcasted_iota(jnp.int32, (BI, 1), 0)

    occ_cols = [jnp.zeros((BI, 1), jnp.float32) for _ in range(NB)]
    for c in range(NCHUNK):
        xj = obs_t_ref[0:1, c * BJ:(c + 1) * BJ]
        yj = obs_t_ref[1:2, c * BJ:(c + 1) * BJ]
        ox = (xj - xi) + (NG / 2.0)
        oy = (yj - yi) + (NG / 2.0)
        j_glob = c * BJ + jax.lax.broadcasted_iota(jnp.int32, (1, BJ), 1)
        valid = ((jnp.minimum(ox, oy) >= 0.0) &
                 (jnp.maximum(ox, oy) < float(NG)) &
                 (i_glob != j_glob))
        xb = ox.astype(jnp.int32)
        yb = oy.astype(jnp.int32)
        binv = jnp.where(valid, xb * NG + yb, -1)
        for k in range(NB):
            hit = jnp.where(binv == k, 1.0, 0.0)
            occ_cols[k] = occ_cols[k] + jnp.sum(hit, axis=1, keepdims=True)

    occ_cols += [jnp.zeros((BI, 1), jnp.float32)] * (ROWW - NB)
    occ = jnp.concatenate(occ_cols, axis=1)
    out_ref[...] = (
        jnp.dot(occ, wt_ref[...], preferred_element_type=jnp.float32)
        + b_ref[...]
    )


def _hist_body(xs_hbm, ys_hbm, occ_hbm, xs_v, ys_v, occ_v, sem):
    cid = lax.axis_index("core")
    sid = lax.axis_index("subcore")
    wid = cid * NS + sid
    base_i = SPLIT + wid * IPW

    copy_x = pltpu.make_async_copy(xs_hbm, xs_v, sem)
    copy_x.start()
    copy_y = pltpu.make_async_copy(ys_hbm, ys_v, sem)
    copy_y.start()

    zero16 = jnp.zeros((LN,), jnp.float32)

    @pl.loop(0, IPW * ROWW, step=LN)
    def _(k):
        occ_v[pl.ds(k, LN)] = zero16

    copy_x.wait()
    copy_y.wait()

    lane = lax.broadcasted_iota(jnp.int32, (LN,), 0)
    ones = jnp.ones((LN,), jnp.float32)

    @pl.loop(0, IPW, step=LN)
    def _(ic):
        xi = xs_v[pl.ds(base_i + ic, LN)]
        yi = ys_v[pl.ds(base_i + ic, LN)]
        rowbase = (ic + lane) * ROWW

        @pl.loop(0, N, step=LN)
        def _(jc):
            xjv = xs_v[pl.ds(jc, LN)]
            yjv = ys_v[pl.ds(jc, LN)]
            for jj in range(LN):
                ox = (xjv[jj] - xi) + (NG / 2.0)
                oy = (yjv[jj] - yi) + (NG / 2.0)
                m = ((jnp.minimum(ox, oy) >= 0.0) &
                     (jnp.maximum(ox, oy) < float(NG)))
                xb = ox.astype(jnp.int32)
                yb = oy.astype(jnp.int32)
                idx = rowbase + (xb * NG + yb)
                plsc.addupdate_scatter(occ_v, [idx], ones, mask=m)

        selfm = (xi * 0.0 == 0.0) & (yi * 0.0 == 0.0)
        plsc.addupdate_scatter(occ_v, [rowbase + (3 * NG + 3)], -ones, mask=selfm)

    pltpu.sync_copy(occ_v, occ_hbm.at[pl.ds(wid * IPW * ROWW, IPW * ROWW)])


def _occupancy_sc(xs, ys):
    mesh = plsc.VectorSubcoreMesh(core_axis_name="core", subcore_axis_name="subcore")
    k = pl.kernel(
        _hist_body,
        out_type=jax.ShapeDtypeStruct((NSC * ROWW,), jnp.float32),
        mesh=mesh,
        scratch_types=[
            pltpu.VMEM((N,), jnp.float32),
            pltpu.VMEM((N,), jnp.float32),
            pltpu.VMEM((IPW * ROWW,), jnp.float32),
            pltpu.SemaphoreType.DMA,
        ],
        compiler_params=_cp,
    )
    return k(xs, ys)


def _embed_kernel(occ_ref, wt_ref, b_ref, out_ref):
    out_ref[...] = (
        jnp.dot(occ_ref[...], wt_ref[...], preferred_element_type=jnp.float32)
        + b_ref[...]
    )


@jax.jit
def kernel(hidden_state, obs1, obs2, W, b):
    del hidden_state, obs1
    obs_t = obs2.T
    wt = jnp.zeros((ROWW, HD), jnp.float32).at[:NB].set(W.T)
    b_row = b.reshape(1, HD)

    occ_sc = _occupancy_sc(obs2[:, 0], obs2[:, 1]).reshape(NSC, ROWW)

    out_tc = pl.pallas_call(
        _pool_kernel,
        grid=(SPLIT // BI,),
        in_specs=[
            pl.BlockSpec((BI, 2), lambda i: (i, 0)),
            pl.BlockSpec((2, N), lambda i: (0, 0)),
            pl.BlockSpec((ROWW, HD), lambda i: (0, 0)),
            pl.BlockSpec((1, HD), lambda i: (0, 0)),
        ],
        out_specs=pl.BlockSpec((BI, HD), lambda i: (i, 0)),
        out_shape=jax.ShapeDtypeStruct((SPLIT, HD), jnp.float32),
    )(obs2[:SPLIT], obs_t, wt, b_row)

    out_sc = pl.pallas_call(
        _embed_kernel,
        grid=(NSC // BM,),
        in_specs=[
            pl.BlockSpec((BM, ROWW), lambda i: (i, 0)),
            pl.BlockSpec((ROWW, HD), lambda i: (0, 0)),
            pl.BlockSpec((1, HD), lambda i: (0, 0)),
        ],
        out_specs=pl.BlockSpec((BM, HD), lambda i: (i, 0)),
        out_shape=jax.ShapeDtypeStruct((NSC, HD), jnp.float32),
    )(occ_sc, wt, b_row)

    return jnp.concatenate([out_tc, out_sc], axis=0)


# speedup vs baseline: 191.7933x; 1.2984x over previous
import dataclasses
import jax
import jax.numpy as jnp
from jax import lax
from jax.experimental import pallas as pl
from jax.experimental.pallas import tpu as pltpu
from jax.experimental.pallas import tpu_sc as plsc

N = 4096
NG = 6
NB = NG * NG
ROWW = 40
HD = 128
NC = 2
NS = 16
NW = NC * NS
LN = 16
BM = 512

SPLIT = 2560
NSC = N - SPLIT
IPW = NSC // NW

BI = 128
BJ = 512
NCHUNK = N // BJ

_cp = pltpu.CompilerParams()
if "needs_layout_passes" in pltpu.CompilerParams.__dataclass_fields__:
    _cp = dataclasses.replace(_cp, needs_layout_passes=False)


def _pool_kernel(obs_i_ref, obs_t_ref, wt_ref, b_ref, out_ref):
    pid = pl.program_id(0)
    xi = obs_i_ref[:, 0:1]
    yi = obs_i_ref[:, 1:2]
    i_glob = pid * BI + jax.lax.broadcasted_iota(jnp.int32, (BI, 1), 0)

    NGRP = (NB + 7) // 8
    accs = [jnp.zeros((BI, BJ), jnp.int32) for _ in range(NGRP)]
    for c in range(NCHUNK):
        xj = obs_t_ref[0:1, c * BJ:(c + 1) * BJ]
        yj = obs_t_ref[1:2, c * BJ:(c + 1) * BJ]
        ox = (xj - xi) + (NG / 2.0)
        oy = (yj - yi) + (NG / 2.0)
        j_glob = c * BJ + jax.lax.broadcasted_iota(jnp.int32, (1, BJ), 1)
        valid = ((jnp.minimum(ox, oy) >= 0.0) &
                 (jnp.maximum(ox, oy) < float(NG)) &
                 (i_glob != j_glob))
        xb = ox.astype(jnp.int32)
        yb = oy.astype(jnp.int32)
        binv = jnp.where(valid, xb * NG + yb, -1)
        w = jnp.left_shift(1, jnp.left_shift(binv & 7, 2))
        g = jnp.right_shift(binv, 3)
        for gi in range(NGRP):
            accs[gi] = accs[gi] + jnp.where(g == gi, w, 0)

    occ_cols = []
    for k in range(NB):
        gi, r = k // 8, k % 8
        nib = jnp.right_shift(accs[gi], 4 * r) & 15
        occ_cols.append(jnp.sum(nib.astype(jnp.float32), axis=1, keepdims=True))
    occ_cols += [jnp.zeros((BI, 1), jnp.float32)] * (ROWW - NB)
    occ = jnp.concatenate(occ_cols, axis=1)
    out_ref[...] = (
        jnp.dot(occ, wt_ref[...], preferred_element_type=jnp.float32)
        + b_ref[...]
    )


def _hist_body(xs_hbm, ys_hbm, occ_hbm, xs_v, ys_v, occ_v, sem):
    cid = lax.axis_index("core")
    sid = lax.axis_index("subcore")
    wid = cid * NS + sid
    base_i = SPLIT + wid * IPW

    copy_x = pltpu.make_async_copy(xs_hbm, xs_v, sem)
    copy_x.start()
    copy_y = pltpu.make_async_copy(ys_hbm, ys_v, sem)
    copy_y.start()

    zero16 = jnp.zeros((LN,), jnp.float32)

    @pl.loop(0, IPW * ROWW, step=LN)
    def _(k):
        occ_v[pl.ds(k, LN)] = zero16

    copy_x.wait()
    copy_y.wait()

    lane = lax.broadcasted_iota(jnp.int32, (LN,), 0)
    ones = jnp.ones((LN,), jnp.float32)

    @pl.loop(0, IPW, step=LN)
    def _(ic):
        xi = xs_v[pl.ds(base_i + ic, LN)]
        yi = ys_v[pl.ds(base_i + ic, LN)]
        rowbase = (ic + lane) * ROWW

        @pl.loop(0, N, step=LN)
        def _(jc):
            xjv = xs_v[pl.ds(jc, LN)]
            yjv = ys_v[pl.ds(jc, LN)]
            for jj in range(LN):
                ox = (xjv[jj] - xi) + (NG / 2.0)
                oy = (yjv[jj] - yi) + (NG / 2.0)
                m = ((jnp.minimum(ox, oy) >= 0.0) &
                     (jnp.maximum(ox, oy) < float(NG)))
                xb = ox.astype(jnp.int32)
                yb = oy.astype(jnp.int32)
                idx = rowbase + (xb * NG + yb)
                plsc.addupdate_scatter(occ_v, [idx], ones, mask=m)

        selfm = (xi * 0.0 == 0.0) & (yi * 0.0 == 0.0)
        plsc.addupdate_scatter(occ_v, [rowbase + (3 * NG + 3)], -ones, mask=selfm)

    pltpu.sync_copy(occ_v, occ_hbm.at[pl.ds(wid * IPW * ROWW, IPW * ROWW)])


def _occupancy_sc(xs, ys):
    mesh = plsc.VectorSubcoreMesh(core_axis_name="core", subcore_axis_name="subcore")
    k = pl.kernel(
        _hist_body,
        out_type=jax.ShapeDtypeStruct((NSC * ROWW,), jnp.float32),
        mesh=mesh,
        scratch_types=[
            pltpu.VMEM((N,), jnp.float32),
            pltpu.VMEM((N,), jnp.float32),
            pltpu.VMEM((IPW * ROWW,), jnp.float32),
            pltpu.SemaphoreType.DMA,
        ],
        compiler_params=_cp,
    )
    return k(xs, ys)


def _embed_kernel(occ_ref, wt_ref, b_ref, out_ref):
    out_ref[...] = (
        jnp.dot(occ_ref[...], wt_ref[...], preferred_element_type=jnp.float32)
        + b_ref[...]
    )


@jax.jit
def kernel(hidden_state, obs1, obs2, W, b):
    del hidden_state, obs1
    obs_t = obs2.T
    wt = jnp.zeros((ROWW, HD), jnp.float32).at[:NB].set(W.T)
    b_row = b.reshape(1, HD)

    occ_sc = _occupancy_sc(obs2[:, 0], obs2[:, 1]).reshape(NSC, ROWW)

    out_tc = pl.pallas_call(
        _pool_kernel,
        grid=(SPLIT // BI,),
        in_specs=[
            pl.BlockSpec((BI, 2), lambda i: (i, 0)),
            pl.BlockSpec((2, N), lambda i: (0, 0)),
            pl.BlockSpec((ROWW, HD), lambda i: (0, 0)),
            pl.BlockSpec((1, HD), lambda i: (0, 0)),
        ],
        out_specs=pl.BlockSpec((BI, HD), lambda i: (i, 0)),
        out_shape=jax.ShapeDtypeStruct((SPLIT, HD), jnp.float32),
    )(obs2[:SPLIT], obs_t, wt, b_row)

    out_sc = pl.pallas_call(
        _embed_kernel,
        grid=(NSC // BM,),
        in_specs=[
            pl.BlockSpec((BM, ROWW), lambda i: (i, 0)),
            pl.BlockSpec((ROWW, HD), lambda i: (0, 0)),
            pl.BlockSpec((1, HD), lambda i: (0, 0)),
        ],
        out_specs=pl.BlockSpec((BM, HD), lambda i: (i, 0)),
        out_shape=jax.ShapeDtypeStruct((NSC, HD), jnp.float32),
    )(occ_sc, wt, b_row)

    return jnp.concatenate([out_tc, out_sc], axis=0)


# speedup vs baseline: 200.0299x; 1.0429x over previous
import dataclasses
import jax
import jax.numpy as jnp
from jax import lax
from jax.experimental import pallas as pl
from jax.experimental.pallas import tpu as pltpu
from jax.experimental.pallas import tpu_sc as plsc

N = 4096
NG = 6
NB = NG * NG
ROWW = 40
HD = 128
NC = 2
NS = 16
NW = NC * NS
LN = 16
BM = 512

SPLIT = 2560
NSC = N - SPLIT
IPW = NSC // NW

BI = 128
BJ = 512
NCHUNK = N // BJ

_cp = pltpu.CompilerParams()
if "needs_layout_passes" in pltpu.CompilerParams.__dataclass_fields__:
    _cp = dataclasses.replace(_cp, needs_layout_passes=False)


def _pool_kernel(obs_i_ref, obs_t_ref, wt_ref, b_ref, out_ref):
    pid = pl.program_id(0)
    xi = obs_i_ref[:, 0:1]
    yi = obs_i_ref[:, 1:2]
    i_glob = pid * BI + jax.lax.broadcasted_iota(jnp.int32, (BI, 1), 0)

    NGRP = (NB + 7) // 8
    accs = [jnp.zeros((BI, BJ), jnp.int32) for _ in range(NGRP)]
    for c in range(NCHUNK):
        xj = obs_t_ref[0:1, c * BJ:(c + 1) * BJ]
        yj = obs_t_ref[1:2, c * BJ:(c + 1) * BJ]
        ox = (xj - xi) + (NG / 2.0)
        oy = (yj - yi) + (NG / 2.0)
        valid = ((jnp.minimum(ox, oy) >= 0.0) &
                 (jnp.maximum(ox, oy) < float(NG)))
        xb = ox.astype(jnp.int32)
        yb = oy.astype(jnp.int32)
        binv = jnp.where(valid, xb * NG + yb, -1)
        w = jnp.left_shift(1, jnp.left_shift(binv & 7, 2))
        g = jnp.right_shift(binv, 3)
        for gi in range(NGRP):
            accs[gi] = accs[gi] + jnp.where(g == gi, w, 0)

    selfhit = jnp.where((xi * 0.0 == 0.0) & (yi * 0.0 == 0.0), 1.0, 0.0)
    occ_cols = []
    for k in range(NB):
        gi, r = k // 8, k % 8
        nib = jnp.right_shift(accs[gi], 4 * r) & 15
        col = jnp.sum(nib.astype(jnp.float32), axis=1, keepdims=True)
        if k == 3 * NG + 3:
            col = col - selfhit
        occ_cols.append(col)
    occ_cols += [jnp.zeros((BI, 1), jnp.float32)] * (ROWW - NB)
    occ = jnp.concatenate(occ_cols, axis=1)
    out_ref[...] = (
        jnp.dot(occ, wt_ref[...], preferred_element_type=jnp.float32)
        + b_ref[...]
    )


def _hist_body(obs_t_hbm, occ_hbm, xs_v, ys_v, occ_v, sem):
    cid = lax.axis_index("core")
    sid = lax.axis_index("subcore")
    wid = cid * NS + sid
    base_i = SPLIT + wid * IPW

    copy_x = pltpu.make_async_copy(obs_t_hbm.at[0], xs_v, sem)
    copy_x.start()
    copy_y = pltpu.make_async_copy(obs_t_hbm.at[1], ys_v, sem)
    copy_y.start()

    zero16 = jnp.zeros((LN,), jnp.float32)

    @pl.loop(0, IPW * ROWW, step=LN)
    def _(k):
        occ_v[pl.ds(k, LN)] = zero16

    copy_x.wait()
    copy_y.wait()

    lane = lax.broadcasted_iota(jnp.int32, (LN,), 0)
    ones = jnp.ones((LN,), jnp.float32)

    @pl.loop(0, IPW, step=LN)
    def _(ic):
        xi = xs_v[pl.ds(base_i + ic, LN)]
        yi = ys_v[pl.ds(base_i + ic, LN)]
        rowbase = (ic + lane) * ROWW

        @pl.loop(0, N, step=LN)
        def _(jc):
            xjv = xs_v[pl.ds(jc, LN)]
            yjv = ys_v[pl.ds(jc, LN)]
            for jj in range(LN):
                ox = (xjv[jj] - xi) + (NG / 2.0)
                oy = (yjv[jj] - yi) + (NG / 2.0)
                m = ((jnp.minimum(ox, oy) >= 0.0) &
                     (jnp.maximum(ox, oy) < float(NG)))
                xb = ox.astype(jnp.int32)
                yb = oy.astype(jnp.int32)
                idx = rowbase + (xb * NG + yb)
                plsc.addupdate_scatter(occ_v, [idx], ones, mask=m)

        selfm = (xi * 0.0 == 0.0) & (yi * 0.0 == 0.0)
        plsc.addupdate_scatter(occ_v, [rowbase + (3 * NG + 3)], -ones, mask=selfm)

    pltpu.sync_copy(occ_v, occ_hbm.at[pl.ds(wid * IPW * ROWW, IPW * ROWW)])


def _occupancy_sc(obs_t):
    mesh = plsc.VectorSubcoreMesh(core_axis_name="core", subcore_axis_name="subcore")
    k = pl.kernel(
        _hist_body,
        out_type=jax.ShapeDtypeStruct((NSC * ROWW,), jnp.float32),
        mesh=mesh,
        scratch_types=[
            pltpu.VMEM((N,), jnp.float32),
            pltpu.VMEM((N,), jnp.float32),
            pltpu.VMEM((IPW * ROWW,), jnp.float32),
            pltpu.SemaphoreType.DMA,
        ],
        compiler_params=_cp,
    )
    return k(obs_t)


def _embed_kernel(prev_ref, occ_ref, wt_ref, b_ref, out_ref):
    del prev_ref
    out_ref[...] = (
        jnp.dot(occ_ref[...], wt_ref[...], preferred_element_type=jnp.float32)
        + b_ref[...]
    )


@jax.jit
def kernel(hidden_state, obs1, obs2, W, b):
    del hidden_state, obs1
    obs_t = obs2.T
    wt = jnp.zeros((ROWW, HD), jnp.float32).at[:NB].set(W.T)
    b_row = b.reshape(1, HD)

    occ_sc = _occupancy_sc(obs_t).reshape(NSC, ROWW)

    out_tc = pl.pallas_call(
        _pool_kernel,
        grid=(SPLIT // BI,),
        in_specs=[
            pl.BlockSpec((BI, 2), lambda i: (i, 0)),
            pl.BlockSpec((2, N), lambda i: (0, 0)),
            pl.BlockSpec((ROWW, HD), lambda i: (0, 0)),
            pl.BlockSpec((1, HD), lambda i: (0, 0)),
        ],
        out_specs=pl.BlockSpec((BI, HD), lambda i: (i, 0)),
        out_shape=jax.ShapeDtypeStruct((N, HD), jnp.float32),
    )(obs2[:SPLIT], obs_t, wt, b_row)

    out = pl.pallas_call(
        _embed_kernel,
        grid=(NSC // BM,),
        in_specs=[
            pl.BlockSpec((BM, HD), lambda i: (i + SPLIT // BM, 0)),
            pl.BlockSpec((BM, ROWW), lambda i: (i, 0)),
            pl.BlockSpec((ROWW, HD), lambda i: (0, 0)),
            pl.BlockSpec((1, HD), lambda i: (0, 0)),
        ],
        out_specs=pl.BlockSpec((BM, HD), lambda i: (i + SPLIT // BM, 0)),
        out_shape=jax.ShapeDtypeStruct((N, HD), jnp.float32),
        input_output_aliases={0: 0},
    )(out_tc, occ_sc, wt, b_row)

    return out
